# Initial kernel scaffold; baseline (speedup 1.0000x reference)
import jax
import jax.numpy as jnp
from jax.experimental import pallas as pl


def kernel(x, edge_index, edge_attr, fw1_1, fb1_1, fw2_1, fb2_1, root_1, bias_1, gamma_1, beta_1, fw1_2, fb1_2, fw2_2, fb2_2, root_2, bias_2, gamma_2, beta_2, fw1_3, fb1_3, fw2_3, fb2_3, root_3, bias_3, gamma_3, beta_3):
    raise NotImplementedError("write your pallas kernel here")


# speedup vs baseline: 4.1185x; 4.1185x over previous
import functools

import jax
import jax.numpy as jnp
from jax import lax
from jax.experimental import pallas as pl
from jax.experimental.pallas import tpu as pltpu
from jax.experimental.pallas import tpu_sc as plsc

N = 10000
E = 160000
DIN = 128
DEMB = 16
EPS = 1e-5
NREG = 17
TW = NREG * 2 * DEMB
NC, NS, L = 2, 16, 16
NW = NC * NS
CH = 128
NCHUNKS = E // CH
CPW = (NCHUNKS + NW - 1) // NW
RPT = 624
REM = N - RPT * NS
BN = 1000
GRID = N // BN


def _prep1_body(x_ref, w_ref, tab_ref, rt_ref):
    o = jnp.dot(x_ref[...], w_ref[...], preferred_element_type=jnp.float32,
                precision=lax.Precision.HIGHEST)
    tab_ref[...] = o[:, :TW]
    rt_ref[...] = o[:, TW:]


def _prep_body(agg_ref, rtin_ref, cvec_ref, w_ref, tab_ref, rt_ref):
    pre = agg_ref[0] + agg_ref[1] + rtin_ref[...] + cvec_ref[0][None, :]
    x = jnp.maximum(pre, 0.0) * cvec_ref[1][None, :] + cvec_ref[2][None, :]
    o = jnp.dot(x, w_ref[...], preferred_element_type=jnp.float32,
                precision=lax.Precision.HIGHEST)
    tab_ref[...] = o[:, :TW]
    rt_ref[...] = o[:, TW:]


def _final_body(agg_ref, rtin_ref, cvec_ref, out_ref):
    pre = agg_ref[0] + agg_ref[1] + rtin_ref[...] + cvec_ref[0][None, :]
    out_ref[...] = jnp.maximum(pre, 0.0) * cvec_ref[1][None, :] + cvec_ref[2][None, :]


def _tc_prep1(x, wcat):
    return pl.pallas_call(
        _prep1_body,
        grid=(GRID,),
        in_specs=[
            pl.BlockSpec((BN, DIN), lambda i: (i, 0)),
            pl.BlockSpec((DIN, TW + DEMB), lambda i: (0, 0)),
        ],
        out_specs=[
            pl.BlockSpec((BN, TW), lambda i: (i, 0)),
            pl.BlockSpec((BN, DEMB), lambda i: (i, 0)),
        ],
        out_shape=[
            jax.ShapeDtypeStruct((N, TW), jnp.float32),
            jax.ShapeDtypeStruct((N, DEMB), jnp.float32),
        ],
    )(x, wcat)


def _tc_prep(aggp, rtin, cvec, wcat):
    return pl.pallas_call(
        _prep_body,
        grid=(GRID,),
        in_specs=[
            pl.BlockSpec((2, BN, DEMB), lambda i: (0, i, 0)),
            pl.BlockSpec((BN, DEMB), lambda i: (i, 0)),
            pl.BlockSpec((3, DEMB), lambda i: (0, 0)),
            pl.BlockSpec((DEMB, TW + DEMB), lambda i: (0, 0)),
        ],
        out_specs=[
            pl.BlockSpec((BN, TW), lambda i: (i, 0)),
            pl.BlockSpec((BN, DEMB), lambda i: (i, 0)),
        ],
        out_shape=[
            jax.ShapeDtypeStruct((N, TW), jnp.float32),
            jax.ShapeDtypeStruct((N, DEMB), jnp.float32),
        ],
    )(aggp, rtin, cvec, wcat)


def _tc_final(aggp, rtin, cvec):
    return pl.pallas_call(
        _final_body,
        grid=(GRID,),
        in_specs=[
            pl.BlockSpec((2, BN, DEMB), lambda i: (0, i, 0)),
            pl.BlockSpec((BN, DEMB), lambda i: (i, 0)),
            pl.BlockSpec((3, DEMB), lambda i: (0, 0)),
        ],
        out_specs=pl.BlockSpec((BN, DEMB), lambda i: (i, 0)),
        out_shape=jax.ShapeDtypeStruct((N, DEMB), jnp.float32),
    )(aggp, rtin, cvec)


def _sc_body(tab_hbm, src_hbm, dst_hbm, ea_hbm, earep_hbm, taub_hbm, out_hbm,
             srcv, dstv, eav, earv, idxv, rows_v, msg_v, taub_v, shared, sem):
    c = lax.axis_index("c")
    s = lax.axis_index("s")
    w = s * NC + c

    pltpu.sync_copy(taub_hbm, taub_v)

    for e in range(CH):
        msg_v[e, :] = jnp.zeros((L,), jnp.float32)
    for j in range(RPT // 104):
        pltpu.sync_copy(msg_v.at[pl.ds(0, 104), :],
                        shared.at[pl.ds(s * RPT + j * 104, 104), :])

    @pl.when(s == NS - 1)
    def _():
        pltpu.sync_copy(msg_v.at[pl.ds(0, REM), :],
                        shared.at[pl.ds(RPT * NS, REM), :])

    plsc.subcore_barrier()

    def chunk_body(i, carry):
        cid = w + i * NW

        @pl.when(cid < NCHUNKS)
        def _():
            base = cid * CH
            pltpu.sync_copy(src_hbm.at[pl.ds(base, CH)], srcv)
            pltpu.sync_copy(dst_hbm.at[pl.ds(base, CH)], dstv)
            pltpu.sync_copy(ea_hbm.at[pl.ds(base, CH)], eav)
            pltpu.sync_copy(earep_hbm.at[pl.ds(base, CH), :], earv)
            for g in range(CH // L):
                ea16 = eav[pl.ds(g * L, L)]
                s16 = srcv[pl.ds(g * L, L)]
                r16 = jnp.zeros((L,), jnp.int32)
                for j in range(L):
                    r16 = r16 + jnp.where(ea16 >= taub_v[j, :], 1, 0).astype(jnp.int32)
                idxv[pl.ds(g * L, L)] = s16 * NREG + r16
            pltpu.async_copy(tab_hbm.at[idxv], rows_v, sem).wait()
            for e in range(CH):
                msg_v[e, :] = (earv[e, :] * rows_v[e, pl.ds(0, L)]
                               + rows_v[e, pl.ds(L, L)])
            pltpu.sync_copy(msg_v, shared.at[dstv], add=True)
        return carry

    lax.fori_loop(0, CPW, chunk_body, 0)
    plsc.subcore_barrier()
    pltpu.sync_copy(shared.at[pl.ds(s * RPT, RPT), :],
                    out_hbm.at[c, pl.ds(s * RPT, RPT), :])

    @pl.when(s == NS - 1)
    def _():
        pltpu.sync_copy(shared.at[pl.ds(RPT * NS, REM), :],
                        out_hbm.at[c, pl.ds(RPT * NS, REM), :])


_SC_MESH = plsc.VectorSubcoreMesh(core_axis_name="c", subcore_axis_name="s",
                                  num_cores=NC, num_subcores=NS)

_sc_layer = pl.kernel(
    _sc_body,
    out_type=jax.ShapeDtypeStruct((2, N, DEMB), jnp.float32),
    mesh=_SC_MESH,
    compiler_params=pltpu.CompilerParams(use_tc_tiling_on_sc=False),
    scratch_types=[
        pltpu.VMEM((CH,), jnp.int32),
        pltpu.VMEM((CH,), jnp.int32),
        pltpu.VMEM((CH,), jnp.float32),
        pltpu.VMEM((CH, L), jnp.float32),
        pltpu.VMEM((CH,), jnp.int32),
        pltpu.VMEM((CH, 2 * L), jnp.float32),
        pltpu.VMEM((CH, L), jnp.float32),
        pltpu.VMEM((L, L), jnp.float32),
        pltpu.VMEM_SHARED((N, DEMB), jnp.float32),
        pltpu.SemaphoreType.DMA,
    ],
)


def _build_layer_weights(fw1, fb1, fw2, fb2, root, din):
    w1 = fw1[0]
    safe = jnp.where(w1 != 0, w1, 1.0)
    t = jnp.where(w1 != 0, -fb1 / safe, jnp.inf)
    tau = jnp.sort(t)
    lo = jnp.concatenate([jnp.array([-1e30], jnp.float32), tau])
    hi = jnp.concatenate([tau, jnp.array([1e30], jnp.float32)])
    rep = (jnp.clip(lo, -9.0, 9.0) + jnp.clip(hi, -9.0, 9.0)) * 0.5
    active = (w1[None, :] * rep[:, None] + fb1[None, :]) > 0
    W2 = fw2.reshape(16, din, DEMB)
    acoef = jnp.where(active, w1[None, :], 0.0)
    bcoef = jnp.where(active, fb1[None, :], 0.0)
    Amat = jnp.einsum('rk,kio->rio', acoef, W2)
    Bmat = jnp.einsum('rk,kio->rio', bcoef, W2) + fb2.reshape(din, DEMB)[None]
    tt = jnp.stack([Amat, Bmat], axis=1)
    wtab = tt.transpose(2, 0, 1, 3).reshape(din, TW)
    wcat = jnp.concatenate([wtab, root], axis=1)
    taub = jnp.broadcast_to(tau[:, None], (L, L))
    return wcat, taub


def _cvec(bias, gamma, beta):
    g = gamma * (1.0 / jnp.sqrt(jnp.float32(1.0 + EPS)))
    return jnp.stack([bias, g, beta * jnp.ones((DEMB,), jnp.float32)])


def kernel(x, edge_index, edge_attr,
           fw1_1, fb1_1, fw2_1, fb2_1, root_1, bias_1, gamma_1, beta_1,
           fw1_2, fb1_2, fw2_2, fb2_2, root_2, bias_2, gamma_2, beta_2,
           fw1_3, fb1_3, fw2_3, fb2_3, root_3, bias_3, gamma_3, beta_3):
    src = edge_index[0]
    dst = edge_index[1]
    ea = edge_attr[:, 0]
    earep = jnp.broadcast_to(edge_attr, (E, L))

    wcat1, taub1 = _build_layer_weights(fw1_1, fb1_1, fw2_1, fb2_1, root_1, DIN)
    wcat2, taub2 = _build_layer_weights(fw1_2, fb1_2, fw2_2, fb2_2, root_2, DEMB)
    wcat3, taub3 = _build_layer_weights(fw1_3, fb1_3, fw2_3, fb2_3, root_3, DEMB)
    cv1 = _cvec(bias_1, gamma_1, beta_1)
    cv2 = _cvec(bias_2, gamma_2, beta_2)
    cv3 = _cvec(bias_3, gamma_3, beta_3)

    tab1, rt1 = _tc_prep1(x, wcat1)
    agg1 = _sc_layer(tab1.reshape(N * NREG, 2 * DEMB), src, dst, ea, earep, taub1)
    tab2, rt2 = _tc_prep(agg1, rt1, cv1, wcat2)
    agg2 = _sc_layer(tab2.reshape(N * NREG, 2 * DEMB), src, dst, ea, earep, taub2)
    tab3, rt3 = _tc_prep(agg2, rt2, cv2, wcat3)
    agg3 = _sc_layer(tab3.reshape(N * NREG, 2 * DEMB), src, dst, ea, earep, taub3)
    return _tc_final(agg3, rt3, cv3)


# speedup vs baseline: 6.3026x; 1.5303x over previous
import functools

import jax
import jax.numpy as jnp
from jax import lax
from jax.experimental import pallas as pl
from jax.experimental.pallas import tpu as pltpu
from jax.experimental.pallas import tpu_sc as plsc

N = 10000
E = 160000
DIN = 128
DEMB = 16
EPS = 1e-5
NREG = 17
TW = NREG * 2 * DEMB
NC, NS, L = 2, 16, 16
NW = NC * NS
CH = 128
SB = 4
IB = SB * CH
NIT = 10
EPW = IB * NIT
EPAD = EPW * NW
NJUNK = 16
RPT = 632
RLAST = N - RPT * (NS - 1)
BN = 1000
GRID = N // BN


def _prep1_body(x_ref, w_ref, tab_ref, rt_ref):
    o = jnp.dot(x_ref[...], w_ref[...], preferred_element_type=jnp.float32,
                precision=lax.Precision.HIGHEST)
    tab_ref[...] = o[:, :TW]
    rt_ref[...] = o[:, TW:]


def _prep_body(agg_ref, rtin_ref, cvec_ref, w_ref, tab_ref, rt_ref):
    pre = agg_ref[0] + agg_ref[1] + rtin_ref[...] + cvec_ref[0][None, :]
    x = jnp.maximum(pre, 0.0) * cvec_ref[1][None, :] + cvec_ref[2][None, :]
    o = jnp.dot(x, w_ref[...], preferred_element_type=jnp.float32,
                precision=lax.Precision.HIGHEST)
    tab_ref[...] = o[:, :TW]
    rt_ref[...] = o[:, TW:]


def _final_body(agg_ref, rtin_ref, cvec_ref, out_ref):
    pre = agg_ref[0] + agg_ref[1] + rtin_ref[...] + cvec_ref[0][None, :]
    out_ref[...] = jnp.maximum(pre, 0.0) * cvec_ref[1][None, :] + cvec_ref[2][None, :]


def _tc_prep1(x, wcat):
    return pl.pallas_call(
        _prep1_body,
        grid=(GRID,),
        in_specs=[
            pl.BlockSpec((BN, DIN), lambda i: (i, 0)),
            pl.BlockSpec((DIN, TW + DEMB), lambda i: (0, 0)),
        ],
        out_specs=[
            pl.BlockSpec((BN, TW), lambda i: (i, 0)),
            pl.BlockSpec((BN, DEMB), lambda i: (i, 0)),
        ],
        out_shape=[
            jax.ShapeDtypeStruct((N, TW), jnp.float32),
            jax.ShapeDtypeStruct((N, DEMB), jnp.float32),
        ],
    )(x, wcat)


def _tc_prep(aggp, rtin, cvec, wcat):
    return pl.pallas_call(
        _prep_body,
        grid=(GRID,),
        in_specs=[
            pl.BlockSpec((2, BN, DEMB), lambda i: (0, i, 0)),
            pl.BlockSpec((BN, DEMB), lambda i: (i, 0)),
            pl.BlockSpec((3, DEMB), lambda i: (0, 0)),
            pl.BlockSpec((DEMB, TW + DEMB), lambda i: (0, 0)),
        ],
        out_specs=[
            pl.BlockSpec((BN, TW), lambda i: (i, 0)),
            pl.BlockSpec((BN, DEMB), lambda i: (i, 0)),
        ],
        out_shape=[
            jax.ShapeDtypeStruct((N, TW), jnp.float32),
            jax.ShapeDtypeStruct((N, DEMB), jnp.float32),
        ],
    )(aggp, rtin, cvec, wcat)


def _tc_final(aggp, rtin, cvec):
    return pl.pallas_call(
        _final_body,
        grid=(GRID,),
        in_specs=[
            pl.BlockSpec((2, BN, DEMB), lambda i: (0, i, 0)),
            pl.BlockSpec((BN, DEMB), lambda i: (i, 0)),
            pl.BlockSpec((3, DEMB), lambda i: (0, 0)),
        ],
        out_specs=pl.BlockSpec((BN, DEMB), lambda i: (i, 0)),
        out_shape=jax.ShapeDtypeStruct((N, DEMB), jnp.float32),
    )(aggp, rtin, cvec)


def _zero_rows(zbuf, shared, nrows, base):
    off = 0
    while off < nrows:
        n = min(CH, nrows - off)
        pltpu.sync_copy(zbuf.at[pl.ds(0, n), :],
                        shared.at[pl.ds(base + off, n), :])
        off += n


def _out_rows(shared, out, c, nrows, base):
    off = 0
    while off < nrows:
        n = min(CH, nrows - off)
        pltpu.sync_copy(shared.at[pl.ds(base + off, n), :],
                        out.at[c, pl.ds(base + off, n), :])
        off += n


def _sc_body(tab_hbm, src_hbm, dst_hbm, ea_hbm, taub_hbm, out_hbm,
             srcv, dstv, eav, idxv, rows_v, msg_v, taub_v, shared,
             isem, gsem, ssem):
    c = lax.axis_index("c")
    s = lax.axis_index("s")
    w = s * NC + c
    ebase = w * EPW

    pltpu.sync_copy(taub_hbm, taub_v)
    tau = [taub_v[j, :] for j in range(L)]

    for e in range(CH):
        msg_v[0, e, :] = jnp.zeros((L,), jnp.float32)

    @pl.when(s < NS - 1)
    def _():
        _zero_rows(msg_v.at[0], shared, RPT, s * RPT)

    @pl.when(s == NS - 1)
    def _():
        _zero_rows(msg_v.at[0], shared, RLAST + NJUNK, s * RPT)

    plsc.subcore_barrier()

    def _issue_inputs(b, it):
        base = ebase + it * IB
        cb = w * (NIT * SB) + it * SB
        pltpu.async_copy(src_hbm.at[pl.ds(base, IB)], srcv.at[b], isem.at[b])
        pltpu.async_copy(dst_hbm.at[pl.ds(cb, SB), :], dstv.at[b], isem.at[b])
        pltpu.async_copy(ea_hbm.at[pl.ds(base, IB)], eav.at[b], isem.at[b])

    def _wait_inputs(b, it):
        base = ebase + it * IB
        cb = w * (NIT * SB) + it * SB
        pltpu.make_async_copy(src_hbm.at[pl.ds(base, IB)], srcv.at[b], isem.at[b]).wait()
        pltpu.make_async_copy(dst_hbm.at[pl.ds(cb, SB), :], dstv.at[b], isem.at[b]).wait()
        pltpu.make_async_copy(ea_hbm.at[pl.ds(base, IB)], eav.at[b], isem.at[b]).wait()

    _issue_inputs(0, 0)

    def iter_body(i, carry):
        b = lax.rem(i, 2)
        _wait_inputs(b, i)

        @pl.when(i < NIT - 1)
        def _():
            _issue_inputs(1 - b, i + 1)

        for j in range(SB):
            def grp_body(g, carry2):
                o = j * CH + g * L
                ea16 = eav[b, pl.ds(o, L)]
                s16 = srcv[b, pl.ds(o, L)]
                r16 = jnp.zeros((L,), jnp.int32)
                for k in range(L):
                    r16 = r16 + jnp.where(ea16 >= tau[k], 1, 0).astype(jnp.int32)
                idxv[j, pl.ds(g * L, L)] = s16 * NREG + r16
                return carry2
            lax.fori_loop(0, CH // L, grp_body, 0)

        gd = [pltpu.async_copy(tab_hbm.at[idxv.at[j]], rows_v.at[j], gsem.at[j])
              for j in range(SB)]

        for j in range(SB):
            @pl.when(i > 0)
            def _():
                pltpu.make_async_copy(msg_v.at[j], shared.at[dstv.at[b, j]],
                                      ssem.at[j]).wait()
            gd[j].wait()

            def msg_body(g, carry2):
                ea16 = eav[b, pl.ds(j * CH + g * L, L)]
                for u in range(L):
                    e = g * L + u
                    msg_v[j, e, :] = (ea16[u] * rows_v[j, e, pl.ds(0, L)]
                                      + rows_v[j, e, pl.ds(L, L)])
                return carry2
            lax.fori_loop(0, CH // L, msg_body, 0)

            pltpu.async_copy(msg_v.at[j], shared.at[dstv.at[b, j]],
                             ssem.at[j], add=True)
        return carry

    lax.fori_loop(0, NIT, iter_body, 0)

    blast = (NIT - 1) % 2
    for j in range(SB):
        pltpu.make_async_copy(msg_v.at[j], shared.at[dstv.at[blast, j]],
                              ssem.at[j]).wait()

    plsc.subcore_barrier()

    @pl.when(s < NS - 1)
    def _():
        _out_rows(shared, out_hbm, c, RPT, s * RPT)

    @pl.when(s == NS - 1)
    def _():
        _out_rows(shared, out_hbm, c, RLAST, s * RPT)


_SC_MESH = plsc.VectorSubcoreMesh(core_axis_name="c", subcore_axis_name="s",
                                  num_cores=NC, num_subcores=NS)

_sc_layer = pl.kernel(
    _sc_body,
    out_type=jax.ShapeDtypeStruct((2, N, DEMB), jnp.float32),
    mesh=_SC_MESH,
    compiler_params=pltpu.CompilerParams(use_tc_tiling_on_sc=False),
    scratch_types=[
        pltpu.VMEM((2, IB), jnp.int32),
        pltpu.VMEM((2, SB, CH), jnp.int32),
        pltpu.VMEM((2, IB), jnp.float32),
        pltpu.VMEM((SB, CH), jnp.int32),
        pltpu.VMEM((SB, CH, 2 * L), jnp.float32),
        pltpu.VMEM((SB, CH, L), jnp.float32),
        pltpu.VMEM((L, L), jnp.float32),
        pltpu.VMEM_SHARED((N + NJUNK, DEMB), jnp.float32),
        pltpu.SemaphoreType.DMA((2,)),
        pltpu.SemaphoreType.DMA((SB,)),
        pltpu.SemaphoreType.DMA((SB,)),
    ],
)


def _build_layer_weights(fw1, fb1, fw2, fb2, root, din):
    w1 = fw1[0]
    safe = jnp.where(w1 != 0, w1, 1.0)
    t = jnp.where(w1 != 0, -fb1 / safe, jnp.inf)
    tau = jnp.sort(t)
    lo = jnp.concatenate([jnp.array([-1e30], jnp.float32), tau])
    hi = jnp.concatenate([tau, jnp.array([1e30], jnp.float32)])
    rep = (jnp.clip(lo, -9.0, 9.0) + jnp.clip(hi, -9.0, 9.0)) * 0.5
    active = (w1[None, :] * rep[:, None] + fb1[None, :]) > 0
    W2 = fw2.reshape(16, din, DEMB)
    acoef = jnp.where(active, w1[None, :], 0.0)
    bcoef = jnp.where(active, fb1[None, :], 0.0)
    Amat = jnp.einsum('rk,kio->rio', acoef, W2)
    Bmat = jnp.einsum('rk,kio->rio', bcoef, W2) + fb2.reshape(din, DEMB)[None]
    tt = jnp.stack([Amat, Bmat], axis=1)
    wtab = tt.transpose(2, 0, 1, 3).reshape(din, TW)
    wcat = jnp.concatenate([wtab, root], axis=1)
    taub = jnp.broadcast_to(tau[:, None], (L, L))
    return wcat, taub


def _cvec(bias, gamma, beta):
    g = gamma * (1.0 / jnp.sqrt(jnp.float32(1.0 + EPS)))
    return jnp.stack([bias, g, beta * jnp.ones((DEMB,), jnp.float32)])


def kernel(x, edge_index, edge_attr,
           fw1_1, fb1_1, fw2_1, fb2_1, root_1, bias_1, gamma_1, beta_1,
           fw1_2, fb1_2, fw2_2, fb2_2, root_2, bias_2, gamma_2, beta_2,
           fw1_3, fb1_3, fw2_3, fb2_3, root_3, bias_3, gamma_3, beta_3):
    src = jnp.pad(edge_index[0], (0, EPAD - E))
    dst = jnp.pad(edge_index[1], (0, EPAD - E),
                  constant_values=N).reshape(EPAD // CH, CH)
    ea = jnp.pad(edge_attr[:, 0], (0, EPAD - E))

    wcat1, taub1 = _build_layer_weights(fw1_1, fb1_1, fw2_1, fb2_1, root_1, DIN)
    wcat2, taub2 = _build_layer_weights(fw1_2, fb1_2, fw2_2, fb2_2, root_2, DEMB)
    wcat3, taub3 = _build_layer_weights(fw1_3, fb1_3, fw2_3, fb2_3, root_3, DEMB)
    cv1 = _cvec(bias_1, gamma_1, beta_1)
    cv2 = _cvec(bias_2, gamma_2, beta_2)
    cv3 = _cvec(bias_3, gamma_3, beta_3)

    tab1, rt1 = _tc_prep1(x, wcat1)
    agg1 = _sc_layer(tab1.reshape(N * NREG, 2 * DEMB), src, dst, ea, taub1)
    tab2, rt2 = _tc_prep(agg1, rt1, cv1, wcat2)
    agg2 = _sc_layer(tab2.reshape(N * NREG, 2 * DEMB), src, dst, ea, taub2)
    tab3, rt3 = _tc_prep(agg2, rt2, cv2, wcat3)
    agg3 = _sc_layer(tab3.reshape(N * NREG, 2 * DEMB), src, dst, ea, taub3)
    return _tc_final(agg3, rt3, cv3)
